# baseline (speedup 1.0000x reference)
import jax
import jax.numpy as jnp
import numpy as np
from jax.experimental import pallas as pl
from jax.experimental.pallas import tpu as pltpu

B, N, DIM, E = 4, 2048, 4096, 64
TOP_NUM = 32
TOKEN_TILE = 512
TILES = N // TOKEN_TILE
INT_MIN = np.int32(-2**31)


def _body(x_ref, w_ref, b_ref, gate_ref, xg_ref, keys_ref, pfx_ref):
    bi = pl.program_id(0)
    ti = pl.program_id(1)

    xt = x_ref[0]
    acc = jax.lax.dot_general(
        xt, w_ref[...], (((1,), (1,)), ((), ())),
        preferred_element_type=jnp.float32)
    acc = acc + b_ref[...]

    sl = pl.ds(ti * TOKEN_TILE, TOKEN_TILE)
    xg_ref[bi, sl, :] = acc

    m = jnp.max(acc, axis=-1, keepdims=True)
    e = jnp.exp(acc - m)
    gate_ref[bi, sl, :] = e / jnp.sum(e, axis=-1, keepdims=True)

    i = jax.lax.bitcast_convert_type(acc, jnp.int32)
    keys = jnp.where(i < 0, jnp.bitwise_xor(~i, INT_MIN), i)
    pair = bi // 2

    @pl.when(bi % 2 == 0)
    def _store_lo():
        keys_ref[pair, sl, 0:E] = keys

    @pl.when(bi % 2 == 1)
    def _store_hi():
        keys_ref[pair, sl, E:2 * E] = keys

    ITERS_PER_STEP = 32 // (2 * TILES)

    @pl.when(bi >= 2)
    def _search_pair0():
        s = (bi - 2) * TILES + ti
        k0 = keys_ref[0]
        prefix = jnp.where(s == 0,
                           jnp.zeros((1, 2 * E), jnp.int32),
                           pfx_ref[0:1, :])
        for sub in range(ITERS_PER_STEP):
            j = ITERS_PER_STEP * s + sub
            bitval = jnp.left_shift(jnp.int32(1), 31 - j)
            cand_u = prefix | bitval
            cand_s = cand_u ^ INT_MIN
            cnt = jnp.sum((k0 >= cand_s).astype(jnp.float32),
                          axis=0, keepdims=True)
            prefix = jnp.where(cnt >= float(TOP_NUM), cand_u, prefix)
        pfx_ref[0:1, :] = prefix

    last = jnp.logical_and(bi == B - 1, ti == TILES - 1)

    @pl.when(last)
    def _finalize():
        thr0 = pfx_ref[0:1, :] ^ INT_MIN

        k1 = keys_ref[1]
        prefix = jnp.zeros((1, 2 * E), jnp.int32)
        for bit in range(31, -1, -1):
            bitval = INT_MIN if bit == 31 else np.int32(1 << bit)
            cand_u = prefix | bitval
            cand_s = cand_u ^ INT_MIN
            cnt = jnp.sum((k1 >= cand_s).astype(jnp.float32),
                          axis=0, keepdims=True)
            prefix = jnp.where(cnt >= float(TOP_NUM), cand_u, prefix)
        thr1 = prefix ^ INT_MIN

        for bb in range(B):
            p, lo = bb // 2, (bb % 2) * E
            thr = thr0 if p == 0 else thr1
            msk = keys_ref[p, :, lo:lo + E] >= thr[:, lo:lo + E]
            gate_ref[bb] = gate_ref[bb] * msk.astype(jnp.float32)


@jax.jit
def kernel(x, W, b):
    b2 = b.reshape(1, E)
    grid = (B, TILES)
    gate, xg = pl.pallas_call(
        _body,
        grid=grid,
        in_specs=[
            pl.BlockSpec((1, TOKEN_TILE, DIM), lambda bi, ti: (bi, ti, 0)),
            pl.BlockSpec((E, DIM), lambda bi, ti: (0, 0)),
            pl.BlockSpec((1, E), lambda bi, ti: (0, 0)),
        ],
        out_specs=[
            pl.BlockSpec((B, N, E), lambda bi, ti: (0, 0, 0)),
            pl.BlockSpec((B, N, E), lambda bi, ti: (0, 0, 0)),
        ],
        out_shape=[
            jax.ShapeDtypeStruct((B, N, E), jnp.float32),
            jax.ShapeDtypeStruct((B, N, E), jnp.float32),
        ],
        scratch_shapes=[
            pltpu.VMEM((2, N, 2 * E), jnp.int32),
            pltpu.VMEM((8, 2 * E), jnp.int32),
        ],
    )(x, W, b2)
    return gate, xg


# speedup vs baseline: 1.0482x; 1.0482x over previous
import jax
import jax.numpy as jnp
import numpy as np
from jax.experimental import pallas as pl
from jax.experimental.pallas import tpu as pltpu

B, N, DIM, E = 4, 2048, 4096, 64
TOP_NUM = 32
TOKEN_TILE = 512
TILES = N // TOKEN_TILE
INT_MIN = np.int32(-2**31)
I16_MIN = np.int16(-2**15)


def _tree_count(m):
    n = m.shape[1]
    while n > 16:
        h = n // 2
        m = m[:, :h, :] + m[:, h:, :]
        n = h
    return jnp.sum(m.astype(jnp.int32), axis=1, keepdims=True)


def _body(x_ref, w_ref, b_ref, gate_ref, xg_ref, keys_ref, hi_ref, lo_ref):
    bi = pl.program_id(0)
    ti = pl.program_id(1)

    xt = x_ref[0]
    acc = jax.lax.dot_general(
        xt, w_ref[...], (((1,), (1,)), ((), ())),
        preferred_element_type=jnp.float32)
    acc = acc + b_ref[...]

    sl = pl.ds(ti * TOKEN_TILE, TOKEN_TILE)
    xg_ref[0, sl, :] = acc

    m = jnp.max(acc, axis=-1, keepdims=True)
    e = jnp.exp(acc - m)
    gate_ref[bi, sl, :] = e / jnp.sum(e, axis=-1, keepdims=True)

    i = jax.lax.bitcast_convert_type(acc, jnp.int32)
    keys = jnp.where(i < 0, jnp.bitwise_xor(~i, INT_MIN), i)
    hi = jax.lax.shift_right_arithmetic(keys, 16).astype(jnp.int16)
    lo = keys.astype(jnp.int16) ^ I16_MIN
    pair = bi // 2

    @pl.when(bi % 2 == 0)
    def _store_lo():
        keys_ref[pair, sl, 0:E] = keys
        hi_ref[pair, sl, 0:E] = hi
        lo_ref[pair, sl, 0:E] = lo

    @pl.when(bi % 2 == 1)
    def _store_hi():
        keys_ref[pair, sl, E:2 * E] = keys
        hi_ref[pair, sl, E:2 * E] = hi
        lo_ref[pair, sl, E:2 * E] = lo

    last = jnp.logical_and(bi == B - 1, ti == TILES - 1)

    @pl.when(last)
    def _finalize():
        one = jnp.int16(1)
        zero = jnp.int16(0)
        top = jnp.int32(TOP_NUM)
        bias = np.int32(0x8000)

        hh = hi_ref[...]
        pfx_h = jnp.zeros((2, 1, 2 * E), jnp.int32)
        for bit in range(15, -1, -1):
            cand_u = pfx_h | np.int32(1 << bit)
            cand_s = (cand_u ^ bias).astype(jnp.int16)
            cnt = _tree_count(jnp.where(hh >= cand_s, one, zero))
            pfx_h = jnp.where(cnt >= top, cand_u, pfx_h)
        h_thr = (pfx_h ^ bias).astype(jnp.int16)

        ll = lo_ref[...]
        eq = hh == h_thr
        g = _tree_count(jnp.where(hh > h_thr, one, zero))
        mm = jnp.where(eq, ll, I16_MIN)
        pfx_l = jnp.zeros((2, 1, 2 * E), jnp.int32)
        for bit in range(15, -1, -1):
            cand_u = pfx_l | np.int32(1 << bit)
            cand_s = (cand_u ^ bias).astype(jnp.int16)
            cnt = g + _tree_count(jnp.where(mm >= cand_s, one, zero))
            pfx_l = jnp.where(cnt >= top, cand_u, pfx_l)

        thr = ((pfx_h ^ bias) << 16) | pfx_l

        kk = keys_ref[...]
        for bb in range(B):
            p, lo_c = bb // 2, (bb % 2) * E
            msk = kk[p, :, lo_c:lo_c + E] >= thr[p, :, lo_c:lo_c + E]
            gate_ref[bb] = gate_ref[bb] * msk.astype(jnp.float32)


@jax.jit
def kernel(x, W, b):
    b2 = b.reshape(1, E)
    grid = (B, TILES)
    gate, xg = pl.pallas_call(
        _body,
        grid=grid,
        in_specs=[
            pl.BlockSpec((1, TOKEN_TILE, DIM), lambda bi, ti: (bi, ti, 0)),
            pl.BlockSpec((E, DIM), lambda bi, ti: (0, 0)),
            pl.BlockSpec((1, E), lambda bi, ti: (0, 0)),
        ],
        out_specs=[
            pl.BlockSpec((B, N, E), lambda bi, ti: (0, 0, 0)),
            pl.BlockSpec((1, N, E), lambda bi, ti: (bi, 0, 0)),
        ],
        out_shape=[
            jax.ShapeDtypeStruct((B, N, E), jnp.float32),
            jax.ShapeDtypeStruct((B, N, E), jnp.float32),
        ],
        scratch_shapes=[
            pltpu.VMEM((2, N, 2 * E), jnp.int32),
            pltpu.VMEM((2, N, 2 * E), jnp.int16),
            pltpu.VMEM((2, N, 2 * E), jnp.int16),
        ],
    )(x, W, b2)
    return gate, xg


# speedup vs baseline: 1.0739x; 1.0246x over previous
import jax
import jax.numpy as jnp
import numpy as np
from jax.experimental import pallas as pl
from jax.experimental.pallas import tpu as pltpu

B, N, DIM, E = 4, 2048, 4096, 64
TOP_NUM = 32
TOKEN_TILE = 512
TILES = N // TOKEN_TILE
INT_MIN = np.int32(-2**31)
I16_MIN = np.int16(-2**15)


def _tree_count(m):
    n = m.shape[1]
    while n > 16:
        h = n // 2
        m = m[:, :h, :] + m[:, h:, :]
        n = h
    return jnp.sum(m.astype(jnp.int32), axis=1, keepdims=True)


def _tree_count2(m):
    n = m.shape[0]
    while n > 16:
        h = n // 2
        m = m[:h, :] + m[h:, :]
        n = h
    return jnp.sum(m.astype(jnp.int32), axis=0, keepdims=True)


def _body(x_ref, w_ref, b_ref, gate_ref, xg_ref, keys_ref, hi_ref, lo_ref,
          pfx_ref):
    bi = pl.program_id(0)
    ti = pl.program_id(1)

    xt = x_ref[0]
    acc = jax.lax.dot_general(
        xt, w_ref[...], (((1,), (1,)), ((), ())),
        preferred_element_type=jnp.float32)
    acc = acc + b_ref[...]

    sl = pl.ds(ti * TOKEN_TILE, TOKEN_TILE)
    xg_ref[0, sl, :] = acc

    m = jnp.max(acc, axis=-1, keepdims=True)
    e = jnp.exp(acc - m)
    gate_ref[bi, sl, :] = e / jnp.sum(e, axis=-1, keepdims=True)

    i = jax.lax.bitcast_convert_type(acc, jnp.int32)
    keys = jnp.where(i < 0, jnp.bitwise_xor(~i, INT_MIN), i)
    hi = jax.lax.shift_right_arithmetic(keys, 16).astype(jnp.int16)
    lo = keys.astype(jnp.int16) ^ I16_MIN
    pair = bi // 2

    @pl.when(bi % 2 == 0)
    def _store_lo():
        keys_ref[pair, sl, 0:E] = keys
        hi_ref[pair, sl, 0:E] = hi
        lo_ref[pair, sl, 0:E] = lo

    @pl.when(bi % 2 == 1)
    def _store_hi():
        keys_ref[pair, sl, E:2 * E] = keys
        hi_ref[pair, sl, E:2 * E] = hi
        lo_ref[pair, sl, E:2 * E] = lo

    one = jnp.int16(1)
    zero = jnp.int16(0)
    top = jnp.int32(TOP_NUM)
    bias = np.int32(0x8000)

    P1_ITERS = 16 // TILES

    @pl.when(bi >= 2)
    def _spread():
        s = (bi - 2) * TILES + ti

        @pl.when(s < TILES)
        def _p1():
            hh0 = hi_ref[0]
            pfx = jnp.where(s == 0,
                            jnp.zeros((1, 2 * E), jnp.int32),
                            pfx_ref[0:1])
            for sub in range(P1_ITERS):
                bit = 15 - (P1_ITERS * s + sub)
                cand_u = pfx | jnp.left_shift(jnp.int32(1), bit)
                cand_s = (cand_u ^ bias).astype(jnp.int16)
                cnt = _tree_count2(jnp.where(hh0 >= cand_s, one, zero))
                pfx = jnp.where(cnt >= top, cand_u, pfx)
            pfx_ref[0:1] = pfx

        @pl.when(s == TILES - 1)
        def _mid():
            h_thr = (pfx_ref[0:1] ^ bias).astype(jnp.int16)
            hh0 = hi_ref[0]
            eq = hh0 == h_thr
            pfx_ref[2:3] = _tree_count2(jnp.where(hh0 > h_thr, one, zero))
            lo_ref[0] = jnp.where(eq, lo_ref[0], I16_MIN)

        @pl.when(s >= TILES)
        def _p2():
            mm0 = lo_ref[0]
            g = pfx_ref[2:3]
            pfx = jnp.where(s == TILES,
                            jnp.zeros((1, 2 * E), jnp.int32),
                            pfx_ref[1:2])
            for sub in range(P1_ITERS):
                bit = 15 - (P1_ITERS * (s - TILES) + sub)
                cand_u = pfx | jnp.left_shift(jnp.int32(1), bit)
                cand_s = (cand_u ^ bias).astype(jnp.int16)
                cnt = g + _tree_count2(jnp.where(mm0 >= cand_s, one, zero))
                pfx = jnp.where(cnt >= top, cand_u, pfx)
            pfx_ref[1:2] = pfx

    last = jnp.logical_and(bi == B - 1, ti == TILES - 1)

    @pl.when(last)
    def _finalize():
        hh = hi_ref[1]
        pfx_h = jnp.zeros((1, 2 * E), jnp.int32)
        for bit in range(15, -1, -1):
            cand_u = pfx_h | np.int32(1 << bit)
            cand_s = (cand_u ^ bias).astype(jnp.int16)
            cnt = _tree_count2(jnp.where(hh >= cand_s, one, zero))
            pfx_h = jnp.where(cnt >= top, cand_u, pfx_h)
        h_thr = (pfx_h ^ bias).astype(jnp.int16)

        eq = hh == h_thr
        g = _tree_count2(jnp.where(hh > h_thr, one, zero))
        mm = jnp.where(eq, lo_ref[1], I16_MIN)
        pfx_l = jnp.zeros((1, 2 * E), jnp.int32)
        for bit in range(15, -1, -1):
            cand_u = pfx_l | np.int32(1 << bit)
            cand_s = (cand_u ^ bias).astype(jnp.int16)
            cnt = g + _tree_count2(jnp.where(mm >= cand_s, one, zero))
            pfx_l = jnp.where(cnt >= top, cand_u, pfx_l)

        thr0 = ((pfx_ref[0:1] ^ bias) << 16) | pfx_ref[1:2]
        thr1 = ((pfx_h ^ bias) << 16) | pfx_l

        for bb in range(B):
            p, lo_c = bb // 2, (bb % 2) * E
            thr = thr0 if p == 0 else thr1
            msk = keys_ref[p, :, lo_c:lo_c + E] >= thr[:, lo_c:lo_c + E]
            gate_ref[bb] = gate_ref[bb] * msk.astype(jnp.float32)


@jax.jit
def kernel(x, W, b):
    b2 = b.reshape(1, E)
    grid = (B, TILES)
    gate, xg = pl.pallas_call(
        _body,
        grid=grid,
        in_specs=[
            pl.BlockSpec((1, TOKEN_TILE, DIM), lambda bi, ti: (bi, ti, 0)),
            pl.BlockSpec((E, DIM), lambda bi, ti: (0, 0)),
            pl.BlockSpec((1, E), lambda bi, ti: (0, 0)),
        ],
        out_specs=[
            pl.BlockSpec((B, N, E), lambda bi, ti: (0, 0, 0)),
            pl.BlockSpec((1, N, E), lambda bi, ti: (bi, 0, 0)),
        ],
        out_shape=[
            jax.ShapeDtypeStruct((B, N, E), jnp.float32),
            jax.ShapeDtypeStruct((B, N, E), jnp.float32),
        ],
        scratch_shapes=[
            pltpu.VMEM((2, N, 2 * E), jnp.int32),
            pltpu.VMEM((2, N, 2 * E), jnp.int16),
            pltpu.VMEM((2, N, 2 * E), jnp.int16),
            pltpu.VMEM((8, 2 * E), jnp.int32),
        ],
    )(x, W, b2)
    return gate, xg
